# baseline (speedup 1.0000x reference)
import jax
import jax.numpy as jnp
from jax import lax
from jax.experimental import pallas as pl
from jax.experimental.pallas import tpu as pltpu
from jax.experimental.pallas import tpu_sc as plsc

_VOCAB = 1000000
_D = 32
_B = 4096
_L = 200
_NC = 2
_NS = 16
_NW = _NC * _NS
_TB = _B // _NW
_NSLOT = 8
_NTURN = _L // _NSLOT


def _gather_kernel(idx_hbm, table_hbm, out_hbm, idx_v, cols, rows, trs, *sems):
    wid = lax.axis_index("s") * _NC + lax.axis_index("c")

    gsems = sems[:_NSLOT]
    ssems = sems[_NSLOT:]

    pltpu.sync_copy(idx_hbm.at[pl.ds(wid * _TB, _TB)], idx_v)

    tok16 = lax.iota(jnp.int32, 16)
    d_lo = tok16
    d_hi = tok16 + 16

    def fire(s, l):
        for g in range(_TB // 16):
            t_vec = tok16 + (16 * g)
            vals = plsc.load_gather(
                idx_v, [t_vec, jnp.full((16,), l, jnp.int32)])
            cols[s][pl.ds(16 * g, 16)] = vals
        pltpu.async_copy(table_hbm.at[cols[s]], rows[s], gsems[s])

    def drain_gather(s):
        pltpu.make_async_copy(
            table_hbm.at[pl.ds(0, _TB)], rows[s], gsems[s]).wait()

    def drain_stores(s):
        for r in range(4):
            pltpu.make_async_copy(
                out_hbm.at[0, 0, 0], trs[s].at[pl.ds(8 * r, 8)],
                ssems[s]).wait()

    def transpose(s):
        @pl.loop(0, _TB, unroll=8)
        def _(t):
            lo = rows[s][t, pl.ds(0, 16)]
            hi = rows[s][t, pl.ds(16, 16)]
            t_vec = jnp.full((16,), t, jnp.int32)
            plsc.store_scatter(trs[s], [d_lo, t_vec], lo)
            plsc.store_scatter(trs[s], [d_hi, t_vec], hi)

    for s in range(_NSLOT):
        fire(s, s)

    @pl.loop(0, _NTURN)
    def body(i):
        lbase = _NSLOT * i
        for s in range(_NSLOT):
            drain_gather(s)

            @pl.when(i > 0)
            def _():
                drain_stores(s)

            transpose(s)
            for r in range(4):
                pltpu.async_copy(
                    trs[s].at[pl.ds(8 * r, 8)],
                    out_hbm.at[lbase + s, r, wid],
                    ssems[s],
                )

            @pl.when(i < _NTURN - 1)
            def _():
                fire(s, lbase + s + _NSLOT)

    for s in range(_NSLOT):
        drain_stores(s)


def _body(idx_hbm, table_hbm, out_hbm, idx_v,
          c0, c1, c2, c3, c4, c5, c6, c7,
          r0, r1, r2, r3, r4, r5, r6, r7,
          t0, t1, t2, t3, t4, t5, t6, t7, *sems):
    _gather_kernel(
        idx_hbm, table_hbm, out_hbm, idx_v,
        (c0, c1, c2, c3, c4, c5, c6, c7),
        (r0, r1, r2, r3, r4, r5, r6, r7),
        (t0, t1, t2, t3, t4, t5, t6, t7),
        *sems)


@jax.jit
def _embed_lookup(indices, table):
    mesh = plsc.VectorSubcoreMesh(core_axis_name="c", subcore_axis_name="s")
    out5 = pl.kernel(
        _body,
        out_type=jax.ShapeDtypeStruct((_L, 4, _NW, 8, 128), jnp.float32),
        mesh=mesh,
        scratch_types=(
            [pltpu.VMEM((_TB, _L), jnp.int32)]
            + [pltpu.VMEM((_TB,), jnp.int32) for _ in range(_NSLOT)]
            + [pltpu.VMEM((_TB, _D), jnp.float32) for _ in range(_NSLOT)]
            + [pltpu.VMEM((_D, _TB), jnp.float32) for _ in range(_NSLOT)]
            + [pltpu.SemaphoreType.DMA for _ in range(2 * _NSLOT)]
        ),
        compiler_params=pltpu.CompilerParams(
            use_tc_tiling_on_sc=False, needs_layout_passes=False),
    )(indices, table)
    return out5.transpose(2, 4, 0, 1, 3).reshape(_B, _L, _D)


def kernel(indices, table):
    return _embed_lookup(indices, table)


# speedup vs baseline: 1.1054x; 1.1054x over previous
import jax
import jax.numpy as jnp
from jax import lax
from jax.experimental import pallas as pl
from jax.experimental.pallas import tpu as pltpu
from jax.experimental.pallas import tpu_sc as plsc

_VOCAB = 1000000
_D = 32
_B = 4096
_L = 200
_NC = 2
_NS = 16
_NW = _NC * _NS
_TB = _B // _NW
_NSLOT = 4
_NTURN = _L // _NSLOT


def _gather_kernel(idx_hbm, table_hbm, out_hbm, idx_v, cols, rows, trs, *sems):
    wid = lax.axis_index("s") * _NC + lax.axis_index("c")

    gsems = sems[:_NSLOT]
    ssems = sems[_NSLOT:]

    pltpu.sync_copy(idx_hbm.at[pl.ds(wid * _TB, _TB)], idx_v)

    tok16 = lax.iota(jnp.int32, 16)
    d_lo = tok16
    d_hi = tok16 + 16

    def fire(s, l):
        for g in range(_TB // 16):
            t_vec = tok16 + (16 * g)
            vals = plsc.load_gather(
                idx_v, [t_vec, jnp.full((16,), l, jnp.int32)])
            cols[s][pl.ds(16 * g, 16)] = vals
        pltpu.async_copy(table_hbm.at[cols[s]], rows[s], gsems[s])

    def drain_gather(s):
        pltpu.make_async_copy(
            table_hbm.at[pl.ds(0, _TB)], rows[s], gsems[s]).wait()

    def drain_stores(s):
        for r in range(4):
            pltpu.make_async_copy(
                out_hbm.at[0, 0, 0], trs[s].at[pl.ds(8 * r, 8)],
                ssems[s]).wait()

    def transpose(s):
        for t in range(_TB // 16):
            t_vec = tok16 + (16 * t)
            for dg in range(_D // 8):
                vals = []
                for j in range(8):
                    d = dg * 8 + j
                    vals.append(plsc.load_gather(
                        rows[s], [t_vec, jnp.full((16,), d, jnp.int32)]))
                for j in range(8):
                    d = dg * 8 + j
                    trs[s][d, pl.ds(16 * t, 16)] = vals[j]

    for s in range(_NSLOT):
        fire(s, s)

    @pl.loop(0, _NTURN)
    def body(i):
        lbase = _NSLOT * i
        for s in range(_NSLOT):
            drain_gather(s)

            @pl.when(i > 0)
            def _():
                drain_stores(s)

            transpose(s)
            for r in range(4):
                pltpu.async_copy(
                    trs[s].at[pl.ds(8 * r, 8)],
                    out_hbm.at[lbase + s, r, wid],
                    ssems[s],
                )

            @pl.when(i < _NTURN - 1)
            def _():
                fire(s, lbase + s + _NSLOT)

    for s in range(_NSLOT):
        drain_stores(s)


def _body(idx_hbm, table_hbm, out_hbm, idx_v, *refs):
    n = _NSLOT
    _gather_kernel(
        idx_hbm, table_hbm, out_hbm, idx_v,
        refs[:n], refs[n:2 * n], refs[2 * n:3 * n], *refs[3 * n:])


@jax.jit
def _embed_lookup(indices, table):
    mesh = plsc.VectorSubcoreMesh(core_axis_name="c", subcore_axis_name="s")
    out5 = pl.kernel(
        _body,
        out_type=jax.ShapeDtypeStruct((_L, 4, _NW, 8, 128), jnp.float32),
        mesh=mesh,
        scratch_types=(
            [pltpu.VMEM((_TB, _L), jnp.int32)]
            + [pltpu.VMEM((_TB,), jnp.int32) for _ in range(_NSLOT)]
            + [pltpu.VMEM((_TB, _D), jnp.float32) for _ in range(_NSLOT)]
            + [pltpu.VMEM((_D, _TB), jnp.float32) for _ in range(_NSLOT)]
            + [pltpu.SemaphoreType.DMA for _ in range(2 * _NSLOT)]
        ),
        compiler_params=pltpu.CompilerParams(
            use_tc_tiling_on_sc=False, needs_layout_passes=False),
    )(indices, table)
    return out5.transpose(2, 4, 0, 1, 3).reshape(_B, _L, _D)


def kernel(indices, table):
    return _embed_lookup(indices, table)


# speedup vs baseline: 1.2587x; 1.1387x over previous
import jax
import jax.numpy as jnp
from jax import lax
from jax.experimental import pallas as pl
from jax.experimental.pallas import tpu as pltpu
from jax.experimental.pallas import tpu_sc as plsc

_VOCAB = 1000000
_D = 32
_B = 4096
_L = 200
_NC = 2
_NS = 16
_NW = _NC * _NS
_TB = _B // _NW
_NSLOT = 2
_NTURN = _L // _NSLOT


def _gather_kernel(idx_hbm, table_hbm, out_hbm, idx_v, cols, rows, trs, *sems):
    wid = lax.axis_index("s") * _NC + lax.axis_index("c")

    gsems = sems[:_NSLOT]
    ssems = sems[_NSLOT:]

    pltpu.sync_copy(idx_hbm.at[pl.ds(wid * _TB, _TB)], idx_v)

    tok16 = lax.iota(jnp.int32, 16)
    d_lo = tok16
    d_hi = tok16 + 16

    def fire(s, l):
        for g in range(_TB // 16):
            t_vec = tok16 + (16 * g)
            vals = plsc.load_gather(
                idx_v, [t_vec, jnp.full((16,), l, jnp.int32)])
            cols[s][pl.ds(16 * g, 16)] = vals
        pltpu.async_copy(table_hbm.at[cols[s]], rows[s], gsems[s])

    def drain_gather(s):
        pltpu.make_async_copy(
            table_hbm.at[pl.ds(0, _TB)], rows[s], gsems[s]).wait()

    def drain_stores(s):
        for r in range(4):
            pltpu.make_async_copy(
                out_hbm.at[0, 0, 0], trs[s].at[pl.ds(8 * r, 8)],
                ssems[s]).wait()

    def transpose(s):
        for g in range(_TB // 16):
            t_vec = tok16 + (16 * g)
            for kg in range(_D // 8):
                vals = []
                dvecs = []
                for j in range(8):
                    k = kg * 8 + j
                    d_vec = (t_vec + k) & (_D - 1)
                    dvecs.append(d_vec)
                    vals.append(plsc.load_gather(rows[s], [t_vec, d_vec]))
                for j in range(8):
                    plsc.store_scatter(trs[s], [dvecs[j], t_vec], vals[j])

    for s in range(_NSLOT):
        fire(s, s)

    @pl.loop(0, _NTURN)
    def body(i):
        lbase = _NSLOT * i
        for s in range(_NSLOT):
            drain_gather(s)

            @pl.when(i > 0)
            def _():
                drain_stores(s)

            transpose(s)
            for r in range(4):
                pltpu.async_copy(
                    trs[s].at[pl.ds(8 * r, 8)],
                    out_hbm.at[lbase + s, r, wid],
                    ssems[s],
                )

            @pl.when(i < _NTURN - 1)
            def _():
                fire(s, lbase + s + _NSLOT)

    for s in range(_NSLOT):
        drain_stores(s)


def _body(idx_hbm, table_hbm, out_hbm, idx_v, *refs):
    n = _NSLOT
    _gather_kernel(
        idx_hbm, table_hbm, out_hbm, idx_v,
        refs[:n], refs[n:2 * n], refs[2 * n:3 * n], *refs[3 * n:])


@jax.jit
def _embed_lookup(indices, table):
    mesh = plsc.VectorSubcoreMesh(core_axis_name="c", subcore_axis_name="s")
    out5 = pl.kernel(
        _body,
        out_type=jax.ShapeDtypeStruct((_L, 4, _NW, 8, 128), jnp.float32),
        mesh=mesh,
        scratch_types=(
            [pltpu.VMEM((_TB, _L), jnp.int32)]
            + [pltpu.VMEM((_TB,), jnp.int32) for _ in range(_NSLOT)]
            + [pltpu.VMEM((_TB, _D), jnp.float32) for _ in range(_NSLOT)]
            + [pltpu.VMEM((_D, _TB), jnp.float32) for _ in range(_NSLOT)]
            + [pltpu.SemaphoreType.DMA for _ in range(2 * _NSLOT)]
        ),
        compiler_params=pltpu.CompilerParams(
            use_tc_tiling_on_sc=False, needs_layout_passes=False),
    )(indices, table)
    return out5.transpose(2, 4, 0, 1, 3).reshape(_B, _L, _D)


def kernel(indices, table):
    return _embed_lookup(indices, table)


# speedup vs baseline: 1.4021x; 1.1139x over previous
import jax
import jax.numpy as jnp
from jax import lax
from jax.experimental import pallas as pl
from jax.experimental.pallas import tpu as pltpu
from jax.experimental.pallas import tpu_sc as plsc

_VOCAB = 1000000
_D = 32
_B = 4096
_L = 200
_NC = 2
_NS = 16
_NW = _NC * _NS
_TB = _B // _NW
_NSLOT = 2
_NTURN = _L // _NSLOT


def _gather_kernel(idx_hbm, table_hbm, out_hbm, idx_v, cols, quads, trs, *sems):
    wid = lax.axis_index("s") * _NC + lax.axis_index("c")

    gsems = sems[:_NSLOT]
    ssems = sems[_NSLOT:]

    pltpu.sync_copy(idx_hbm.at[pl.ds(wid * _TB, _TB)], idx_v)

    tok16 = lax.iota(jnp.int32, 16)
    one16 = tok16 * 0 + 1

    def fire(s, l):
        for g in range(_TB // 16):
            t_vec = tok16 + (16 * g)
            vals = plsc.load_gather(
                idx_v, [t_vec, jnp.full((16,), l, jnp.int32)])
            oct0 = vals * 4
            for r in range(4):
                cols[s][r][pl.ds(16 * g, 16)] = oct0 + r
        for r in range(4):
            pltpu.async_copy(
                table_hbm.at[cols[s][r]], quads[s][r], gsems[s])

    def drain_gather(s):
        for r in range(4):
            pltpu.make_async_copy(
                table_hbm.at[pl.ds(0, _TB)], quads[s][r], gsems[s]).wait()

    def drain_stores(s):
        for r in range(4):
            pltpu.make_async_copy(
                out_hbm.at[0, 0, 0], trs[s].at[pl.ds(8 * r, 8)],
                ssems[s]).wait()

    def transpose(s):
        for g in range(_TB // 16):
            t_vec = tok16 + (16 * g)
            for r in range(4):
                k_vec = one16 * 0
                vals = []
                for k in range(8):
                    vals.append(plsc.load_gather(quads[s][r], [t_vec, k_vec]))
                    k_vec = k_vec + 1
                for k in range(8):
                    trs[s][8 * r + k, pl.ds(16 * g, 16)] = vals[k]

    for s in range(_NSLOT):
        fire(s, s)

    @pl.loop(0, _NTURN)
    def body(i):
        lbase = _NSLOT * i
        for s in range(_NSLOT):
            drain_gather(s)

            @pl.when(i > 0)
            def _():
                drain_stores(s)

            transpose(s)
            for r in range(4):
                pltpu.async_copy(
                    trs[s].at[pl.ds(8 * r, 8)],
                    out_hbm.at[lbase + s, r, wid],
                    ssems[s],
                )

            @pl.when(i < _NTURN - 1)
            def _():
                fire(s, lbase + s + _NSLOT)

    for s in range(_NSLOT):
        drain_stores(s)


def _body(idx_hbm, table_hbm, out_hbm, idx_v, *refs):
    n = _NSLOT
    cols = tuple(tuple(refs[4 * s + r] for r in range(4)) for s in range(n))
    quads = tuple(
        tuple(refs[4 * n + 4 * s + r] for r in range(4)) for s in range(n))
    trs = tuple(refs[8 * n + s] for s in range(n))
    _gather_kernel(
        idx_hbm, table_hbm, out_hbm, idx_v, cols, quads, trs,
        *refs[9 * n:])


@jax.jit
def _embed_lookup(indices, table):
    mesh = plsc.VectorSubcoreMesh(core_axis_name="c", subcore_axis_name="s")
    table4 = table.reshape(_VOCAB * 4, 8)
    out5 = pl.kernel(
        _body,
        out_type=jax.ShapeDtypeStruct((_L, 4, _NW, 8, 128), jnp.float32),
        mesh=mesh,
        scratch_types=(
            [pltpu.VMEM((_TB, _L), jnp.int32)]
            + [pltpu.VMEM((_TB,), jnp.int32) for _ in range(4 * _NSLOT)]
            + [pltpu.VMEM((_TB, 8), jnp.float32) for _ in range(4 * _NSLOT)]
            + [pltpu.VMEM((_D, _TB), jnp.float32) for _ in range(_NSLOT)]
            + [pltpu.SemaphoreType.DMA for _ in range(2 * _NSLOT)]
        ),
        compiler_params=pltpu.CompilerParams(
            use_tc_tiling_on_sc=False, needs_layout_passes=False),
    )(indices, table4)
    return out5.transpose(2, 4, 0, 1, 3).reshape(_B, _L, _D)


def kernel(indices, table):
    return _embed_lookup(indices, table)


# speedup vs baseline: 1.4566x; 1.0389x over previous
import jax
import jax.numpy as jnp
from jax import lax
from jax.experimental import pallas as pl
from jax.experimental.pallas import tpu as pltpu
from jax.experimental.pallas import tpu_sc as plsc

_VOCAB = 1000000
_D = 32
_B = 4096
_L = 200
_NC = 2
_NS = 16
_NW = _NC * _NS
_TB = _B // _NW
_NSLOT = 4
_NTURN = _L // _NSLOT


def _gather_kernel(idx_hbm, table_hbm, out_hbm, idx_v, cols, quads, trs, *sems):
    wid = lax.axis_index("s") * _NC + lax.axis_index("c")

    gsems = sems[:_NSLOT]
    ssems = sems[_NSLOT:]

    pltpu.sync_copy(idx_hbm.at[pl.ds(wid * _TB, _TB)], idx_v)

    tok16 = lax.iota(jnp.int32, 16)
    one16 = tok16 * 0 + 1

    def fire(s, l):
        for g in range(_TB // 16):
            t_vec = tok16 + (16 * g)
            vals = plsc.load_gather(
                idx_v, [t_vec, jnp.full((16,), l, jnp.int32)])
            oct0 = vals * 4
            for r in range(4):
                cols[s][r][pl.ds(16 * g, 16)] = oct0 + r
        for r in range(4):
            pltpu.async_copy(
                table_hbm.at[cols[s][r]], quads[s][r], gsems[s])

    def drain_gather(s):
        for r in range(4):
            pltpu.make_async_copy(
                table_hbm.at[pl.ds(0, _TB)], quads[s][r], gsems[s]).wait()

    def drain_stores(s):
        for r in range(4):
            pltpu.make_async_copy(
                out_hbm.at[0, 0, 0], trs[s].at[pl.ds(8 * r, 8)],
                ssems[s]).wait()

    def transpose(s):
        for g in range(_TB // 16):
            t_vec = tok16 + (16 * g)
            for r in range(4):
                k_vec = one16 * 0
                vals = []
                for k in range(8):
                    vals.append(plsc.load_gather(quads[s][r], [t_vec, k_vec]))
                    k_vec = k_vec + 1
                for k in range(8):
                    trs[s][8 * r + k, pl.ds(16 * g, 16)] = vals[k]

    for s in range(_NSLOT):
        fire(s, s)

    @pl.loop(0, _NTURN)
    def body(i):
        lbase = _NSLOT * i
        for s in range(_NSLOT):
            drain_gather(s)

            @pl.when(i > 0)
            def _():
                drain_stores(s)

            transpose(s)
            for r in range(4):
                pltpu.async_copy(
                    trs[s].at[pl.ds(8 * r, 8)],
                    out_hbm.at[lbase + s, r, wid],
                    ssems[s],
                )

            @pl.when(i < _NTURN - 1)
            def _():
                fire(s, lbase + s + _NSLOT)

    for s in range(_NSLOT):
        drain_stores(s)


def _body(idx_hbm, table_hbm, out_hbm, idx_v, *refs):
    n = _NSLOT
    cols = tuple(tuple(refs[4 * s + r] for r in range(4)) for s in range(n))
    quads = tuple(
        tuple(refs[4 * n + 4 * s + r] for r in range(4)) for s in range(n))
    trs = tuple(refs[8 * n + s] for s in range(n))
    _gather_kernel(
        idx_hbm, table_hbm, out_hbm, idx_v, cols, quads, trs,
        *refs[9 * n:])


@jax.jit
def _embed_lookup(indices, table):
    mesh = plsc.VectorSubcoreMesh(core_axis_name="c", subcore_axis_name="s")
    table4 = table.reshape(_VOCAB * 4, 8)
    out5 = pl.kernel(
        _body,
        out_type=jax.ShapeDtypeStruct((_L, 4, _NW, 8, 128), jnp.float32),
        mesh=mesh,
        scratch_types=(
            [pltpu.VMEM((_TB, _L), jnp.int32)]
            + [pltpu.VMEM((_TB,), jnp.int32) for _ in range(4 * _NSLOT)]
            + [pltpu.VMEM((_TB, 8), jnp.float32) for _ in range(4 * _NSLOT)]
            + [pltpu.VMEM((_D, _TB), jnp.float32) for _ in range(_NSLOT)]
            + [pltpu.SemaphoreType.DMA for _ in range(2 * _NSLOT)]
        ),
        compiler_params=pltpu.CompilerParams(
            use_tc_tiling_on_sc=False, needs_layout_passes=False),
    )(indices, table4)
    return out5.transpose(2, 4, 0, 1, 3).reshape(_B, _L, _D)


def kernel(indices, table):
    return _embed_lookup(indices, table)
